# Initial kernel scaffold; baseline (speedup 1.0000x reference)
import jax
import jax.numpy as jnp
from jax.experimental import pallas as pl


def kernel(x, edge_index, W1, b1, g1, be1, a1, p1W, p1b, W2, b2, g2, be2, a2, p2W, p2b):
    raise NotImplementedError("write your pallas kernel here")


# speedup vs baseline: 3.7581x; 3.7581x over previous
import functools

import jax
import jax.numpy as jnp
from jax import lax
from jax.experimental import pallas as pl
from jax.experimental.pallas import tpu as pltpu
from jax.experimental.pallas import tpu_sc as plsc

N1 = 10000
E = 320000
F = 128
K1 = 8000
K2 = 6400
BN_EPS = 1e-5

NC, NS = 2, 16
NW = NC * NS
CH = 128

E_PAD = 323584
EPW = E_PAD // NW
ECH = EPW // CH

S1 = 10112
S2 = 8192
NP1 = 12288
NP2 = 8192
XNEW = 8008
HOUT = 6408

@functools.cache
def _mesh():
    return plsc.VectorSubcoreMesh(
        core_axis_name="c", subcore_axis_name="s", num_cores=NC, num_subcores=NS)


def _wid():
    return lax.axis_index("c") * NS + lax.axis_index("s")


def _fill_ones(ref):
    for t in range(CH // 16):
        ref[pl.ds(t * 16, 16)] = jnp.full((16,), 1.0, jnp.float32)


def _make_sc_deg(s_rows):
    wb = s_rows // NS

    @functools.partial(
        pl.kernel,
        out_type=jax.ShapeDtypeStruct((NC * s_rows,), jnp.float32),
        mesh=_mesh(),
        scratch_types=[
            pltpu.VMEM((CH,), jnp.int32),
            pltpu.VMEM((CH,), jnp.float32),
            pltpu.VMEM(((wb + 15) // 16 * 16,), jnp.float32),
            pltpu.VMEM_SHARED((s_rows,), jnp.float32),
        ],
    )
    def k(dst_hbm, out_hbm, didx, ones, stage, acc):
        c = lax.axis_index("c")
        s = lax.axis_index("s")
        _fill_ones(ones)
        for t in range((wb + 15) // 16):
            stage[pl.ds(t * 16, 16)] = jnp.zeros((16,), jnp.float32)
        pltpu.sync_copy(stage.at[pl.ds(0, wb)], acc.at[pl.ds(s * wb, wb)])
        plsc.subcore_barrier()
        base = _wid() * EPW

        def body(j, carry):
            off = base + j * CH
            pltpu.sync_copy(dst_hbm.at[pl.ds(off, CH)], didx)
            pltpu.sync_copy(ones, acc.at[didx], add=True)
            return carry

        lax.fori_loop(0, ECH, body, 0)
        plsc.subcore_barrier()
        pltpu.sync_copy(acc.at[pl.ds(s * wb, wb)], stage.at[pl.ds(0, wb)])
        pltpu.sync_copy(stage.at[pl.ds(0, wb)],
                        out_hbm.at[pl.ds(c * s_rows + s * wb, wb)])

    return k


def _make_sc_edge_rows(n_rows, s_rows):
    wb = s_rows // NS

    @functools.partial(
        pl.kernel,
        out_type=jax.ShapeDtypeStruct((NC, s_rows, F), jnp.float32),
        mesh=_mesh(),
        scratch_types=[
            pltpu.VMEM((CH,), jnp.int32),
            pltpu.VMEM((CH,), jnp.int32),
            pltpu.VMEM((CH, F), jnp.float32),
            pltpu.VMEM_SHARED((s_rows, F), jnp.float32),
            pltpu.SemaphoreType.DMA,
        ],
    )
    def k(y_hbm, src_hbm, dst_hbm, zero_hbm, out_hbm, sidx, didx, rows, acc, sem):
        c = lax.axis_index("c")
        s = lax.axis_index("s")
        lo = s * wb
        nfull = wb // CH
        for i in range(nfull):
            pltpu.sync_copy(zero_hbm, acc.at[pl.ds(lo + i * CH, CH), :])
        rem = wb - nfull * CH
        if rem:
            pltpu.sync_copy(zero_hbm.at[pl.ds(0, rem), :],
                            acc.at[pl.ds(lo + nfull * CH, rem), :])
        plsc.subcore_barrier()
        base = _wid() * EPW

        def body(j, carry):
            off = base + j * CH
            pltpu.sync_copy(src_hbm.at[pl.ds(off, CH)], sidx)
            pltpu.sync_copy(dst_hbm.at[pl.ds(off, CH)], didx)
            pltpu.async_copy(y_hbm.at[sidx], rows, sem).wait()
            pltpu.sync_copy(rows, acc.at[didx], add=True)
            return carry

        lax.fori_loop(0, ECH, body, 0)
        plsc.subcore_barrier()
        pltpu.sync_copy(acc.at[pl.ds(s * wb, wb), :],
                        out_hbm.at[c, pl.ds(s * wb, wb), :])

    return k


def _make_sc_edge_scal(n_rows, s_rows):
    wb = s_rows // NS

    @functools.partial(
        pl.kernel,
        out_type=jax.ShapeDtypeStruct((NC * s_rows,), jnp.float32),
        mesh=_mesh(),
        scratch_types=[
            pltpu.VMEM((CH,), jnp.int32),
            pltpu.VMEM((CH,), jnp.int32),
            pltpu.VMEM((CH,), jnp.float32),
            pltpu.VMEM(((wb + 15) // 16 * 16,), jnp.float32),
            pltpu.VMEM_SHARED((s_rows,), jnp.float32),
            pltpu.SemaphoreType.DMA,
        ],
    )
    def k(u_hbm, src_hbm, dst_hbm, out_hbm,
          sidx, didx, vals, stage, acc, sem):
        c = lax.axis_index("c")
        s = lax.axis_index("s")
        for t in range((wb + 15) // 16):
            stage[pl.ds(t * 16, 16)] = jnp.zeros((16,), jnp.float32)
        pltpu.sync_copy(stage.at[pl.ds(0, wb)], acc.at[pl.ds(s * wb, wb)])
        plsc.subcore_barrier()
        base = _wid() * EPW

        def body(j, carry):
            off = base + j * CH
            pltpu.sync_copy(src_hbm.at[pl.ds(off, CH)], sidx)
            pltpu.sync_copy(dst_hbm.at[pl.ds(off, CH)], didx)
            pltpu.async_copy(u_hbm.at[sidx], vals, sem).wait()
            pltpu.sync_copy(vals, acc.at[didx], add=True)
            return carry

        lax.fori_loop(0, ECH, body, 0)
        plsc.subcore_barrier()
        pltpu.sync_copy(acc.at[pl.ds(s * wb, wb)], stage.at[pl.ds(0, wb)])
        pltpu.sync_copy(stage.at[pl.ds(0, wb)],
                        out_hbm.at[pl.ds(c * s_rows + s * wb, wb)])

    return k


def _make_sc_pool_remap():
    wb2 = S2 // NS
    nch = NP1 // NW // CH

    @functools.partial(
        pl.kernel,
        out_type=(
            jax.ShapeDtypeStruct((XNEW, F), jnp.float32),
            jax.ShapeDtypeStruct((E_PAD,), jnp.int32),
            jax.ShapeDtypeStruct((E_PAD,), jnp.int32),
            jax.ShapeDtypeStruct((NC * S2,), jnp.float32),
        ),
        mesh=_mesh(),
        scratch_types=[
            pltpu.VMEM((CH,), jnp.int32),
            pltpu.VMEM((CH,), jnp.int32),
            pltpu.VMEM((CH,), jnp.int32),
            pltpu.VMEM((CH,), jnp.int32),
            pltpu.VMEM((CH,), jnp.int32),
            pltpu.VMEM((CH,), jnp.int32),
            pltpu.VMEM((CH, F), jnp.float32),
            pltpu.VMEM((CH,), jnp.float32),
            pltpu.VMEM((wb2,), jnp.float32),
            pltpu.VMEM_SHARED((S2,), jnp.float32),
            pltpu.SemaphoreType.DMA,
            pltpu.SemaphoreType.DMA,
        ],
    )
    def k(xs_hbm, dpool_hbm, nidx_hbm, src1_hbm, dst1_hbm,
          xnew_hbm, src2_hbm, dst2_hbm, dg_hbm,
          sidx, didx, nrow, ncol, s2b, d2b, rows, ones, stage, acc, sem, sem2):
        c = lax.axis_index("c")
        s = lax.axis_index("s")
        _fill_ones(ones)
        for t in range(wb2 // 16):
            stage[pl.ds(t * 16, 16)] = jnp.zeros((16,), jnp.float32)
        pltpu.sync_copy(stage, acc.at[pl.ds(s * wb2, wb2)])
        plsc.subcore_barrier()
        w = _wid()

        nbase = w * (NP1 // NW)

        def nbody(j, carry):
            off = nbase + j * CH
            pltpu.sync_copy(dpool_hbm.at[pl.ds(off, CH)], didx)
            pltpu.sync_copy(xs_hbm.at[pl.ds(off, CH), :], rows)
            pltpu.async_copy(rows, xnew_hbm.at[didx], sem).wait()
            return carry

        lax.fori_loop(0, nch, nbody, 0)

        base = w * EPW

        def ebody(j, carry):
            off = base + j * CH
            pltpu.sync_copy(src1_hbm.at[pl.ds(off, CH)], sidx)
            pltpu.sync_copy(dst1_hbm.at[pl.ds(off, CH)], didx)
            pltpu.async_copy(nidx_hbm.at[sidx], nrow, sem).wait()
            pltpu.async_copy(nidx_hbm.at[didx], ncol, sem2).wait()
            for t in range(CH // 16):
                dd = pl.ds(t * 16, 16)
                nr = nrow[dd]
                nc2 = ncol[dd]
                ok = (nr >= 0) & (nc2 >= 0)
                s2b[dd] = jnp.where(ok, nr, 0)
                d2b[dd] = jnp.where(ok, nc2, K1)
            pltpu.sync_copy(s2b, src2_hbm.at[pl.ds(off, CH)])
            pltpu.sync_copy(d2b, dst2_hbm.at[pl.ds(off, CH)])
            pltpu.sync_copy(ones, acc.at[d2b], add=True)
            return carry

        lax.fori_loop(0, ECH, ebody, 0)
        plsc.subcore_barrier()
        pltpu.sync_copy(acc.at[pl.ds(s * wb2, wb2)], stage)
        pltpu.sync_copy(stage, dg_hbm.at[pl.ds(c * S2 + s * wb2, wb2)])

    return k


def _make_sc_pool(np_dom, out_rows):
    nch = np_dom // NW // CH

    @functools.partial(
        pl.kernel,
        out_type=jax.ShapeDtypeStruct((out_rows, F), jnp.float32),
        mesh=_mesh(),
        scratch_types=[
            pltpu.VMEM((CH,), jnp.int32),
            pltpu.VMEM((CH, F), jnp.float32),
            pltpu.SemaphoreType.DMA,
        ],
    )
    def k(xs_hbm, dpool_hbm, out_hbm, didx, rows, sem):
        base = _wid() * (np_dom // NW)

        def body(j, carry):
            off = base + j * CH
            pltpu.sync_copy(dpool_hbm.at[pl.ds(off, CH)], didx)
            pltpu.sync_copy(xs_hbm.at[pl.ds(off, CH), :], rows)
            pltpu.async_copy(rows, out_hbm.at[didx], sem).wait()
            return carry

        lax.fori_loop(0, nch, body, 0)

    return k


def _tc_dense(x, W, degT, n):
    def body(x_ref, w_ref, d_ref, y_ref, dinv_ref):
        p = d_ref[...]
        dinv = lax.rsqrt(p[:, 0:1] + p[:, 1:2] + 1.0)
        y_ref[...] = jnp.dot(x_ref[...], w_ref[...],
                             preferred_element_type=jnp.float32,
                             precision=lax.Precision.HIGHEST) * dinv
        dinv_ref[...] = dinv

    return pl.pallas_call(
        body,
        grid=(1,),
        in_specs=[
            pl.BlockSpec((n, F), lambda i: (0, 0)),
            pl.BlockSpec((F, F), lambda i: (0, 0)),
            pl.BlockSpec((n, 2), lambda i: (0, 0)),
        ],
        out_specs=[
            pl.BlockSpec((n, F), lambda i: (0, 0)),
            pl.BlockSpec((n, 1), lambda i: (0, 0)),
        ],
        out_shape=[
            jax.ShapeDtypeStruct((n, F), jnp.float32),
            jax.ShapeDtypeStruct((n, 1), jnp.float32),
        ],
    )(x, W, degT)


def _tc_bn(sp, y, dinv, b, g, be, a, pw, n, s_rows):
    def body(sp_ref, y_ref, dv_ref, b_ref, g_ref, be_ref, a_ref, pw_ref,
             hp_ref, u_ref):
        S = sp_ref[0] + sp_ref[1]
        h = (S + y_ref[...]) * dv_ref[...] + b_ref[...]
        mu = jnp.mean(h, axis=0, keepdims=True)
        xc = h - mu
        var = jnp.mean(xc * xc, axis=0, keepdims=True)
        hb = g_ref[...] * xc * lax.rsqrt(var + BN_EPS) + be_ref[...]
        hp = jnp.where(hb >= 0.0, hb, a_ref[...] * hb)
        hp_ref[...] = hp
        u_ref[...] = jnp.dot(hp, pw_ref[...],
                             preferred_element_type=jnp.float32) * dv_ref[...]

    return pl.pallas_call(
        body,
        grid=(1,),
        in_specs=[
            pl.BlockSpec((NC, n, F), lambda i: (0, 0, 0)),
            pl.BlockSpec((n, F), lambda i: (0, 0)),
            pl.BlockSpec((n, 1), lambda i: (0, 0)),
            pl.BlockSpec((1, F), lambda i: (0, 0)),
            pl.BlockSpec((1, F), lambda i: (0, 0)),
            pl.BlockSpec((1, F), lambda i: (0, 0)),
            pl.BlockSpec((1, 1), lambda i: (0, 0)),
            pl.BlockSpec((F, 1), lambda i: (0, 0)),
        ],
        out_specs=[
            pl.BlockSpec((n, F), lambda i: (0, 0)),
            pl.BlockSpec((n, 1), lambda i: (0, 0)),
        ],
        out_shape=[
            jax.ShapeDtypeStruct((n, F), jnp.float32),
            jax.ShapeDtypeStruct((n, 1), jnp.float32),
        ],
    )(sp, y, dinv, b, g, be, a, pw)


def _tc_rank(ssT, u_col, dinv_col, ssR, u_row, dinv_row, pb, hp, n, k, trash,
             s_rows):
    B = 256
    grid = (n + B - 1) // B

    def body(ssT_ref, u_ref, dv_ref, ssR_ref, uR_ref, dvR_ref, pb_ref, hp_ref,
             xs_ref, ni_ref, dp_ref):
        bi = pl.program_id(0)
        s_col = (ssT_ref[:, 0:1] + ssT_ref[:, 1:2] + u_ref[...]) * dv_ref[...] \
            + pb_ref[...]
        s_row = (ssR_ref[0:1, 0:n] + ssR_ref[1:2, 0:n] + uR_ref[...]) \
            * dvR_ref[...] + pb_ref[...]
        gt = s_row > s_col
        eq = s_row == s_col
        jj = lax.broadcasted_iota(jnp.int32, (1, n), 1)
        ii = bi * B + lax.broadcasted_iota(jnp.int32, (B, 1), 0)
        beat = (gt | (eq & (jj < ii))).astype(jnp.int32)
        rank = jnp.sum(beat, axis=1, keepdims=True)
        keep = rank < k
        ni_ref[...] = jnp.where(keep, rank, -1)
        dp_ref[...] = jnp.where(keep, rank, trash)
        coef = jnp.where(keep, jnp.tanh(s_col), 0.0)
        xs_ref[...] = hp_ref[...] * coef

    return pl.pallas_call(
        body,
        grid=(grid,),
        in_specs=[
            pl.BlockSpec((B, 2), lambda i: (i, 0)),
            pl.BlockSpec((B, 1), lambda i: (i, 0)),
            pl.BlockSpec((B, 1), lambda i: (i, 0)),
            pl.BlockSpec((NC, s_rows), lambda i: (0, 0)),
            pl.BlockSpec((1, n), lambda i: (0, 0)),
            pl.BlockSpec((1, n), lambda i: (0, 0)),
            pl.BlockSpec((1, 1), lambda i: (0, 0)),
            pl.BlockSpec((B, F), lambda i: (i, 0)),
        ],
        out_specs=[
            pl.BlockSpec((B, F), lambda i: (i, 0)),
            pl.BlockSpec((B, 1), lambda i: (i, 0)),
            pl.BlockSpec((B, 1), lambda i: (i, 0)),
        ],
        out_shape=[
            jax.ShapeDtypeStruct((n, F), jnp.float32),
            jax.ShapeDtypeStruct((n, 1), jnp.int32),
            jax.ShapeDtypeStruct((n, 1), jnp.int32),
        ],
    )(ssT, u_col, dinv_col, ssR, u_row, dinv_row, pb, hp)


def _tc_mean(hout):
    def body(h_ref, z_ref):
        z_ref[...] = jnp.mean(h_ref[0:K2, :], axis=0, keepdims=True)

    return pl.pallas_call(
        body,
        grid=(1,),
        in_specs=[pl.BlockSpec((HOUT, F), lambda i: (0, 0))],
        out_specs=pl.BlockSpec((1, F), lambda i: (0, 0)),
        out_shape=jax.ShapeDtypeStruct((1, F), jnp.float32),
    )(hout)


@functools.cache
def _built():
    return {
        "deg1": _make_sc_deg(S1),
        "rows1": _make_sc_edge_rows(N1, S1),
        "rows2": _make_sc_edge_rows(K1, S2),
        "scal1": _make_sc_edge_scal(N1, S1),
        "scal2": _make_sc_edge_scal(K1, S2),
        "pool_remap": _make_sc_pool_remap(),
        "pool2": _make_sc_pool(NP2, HOUT),
    }


def _sc_deg1(*a):
    return _built()["deg1"](*a)


def _sc_edge_rows1(*a):
    return _built()["rows1"](*a)


def _sc_edge_rows2(*a):
    return _built()["rows2"](*a)


def _sc_edge_scal1(*a):
    return _built()["scal1"](*a)


def _sc_edge_scal2(*a):
    return _built()["scal2"](*a)


def _sc_pool_remap(*a):
    return _built()["pool_remap"](*a)


def _sc_pool2(*a):
    return _built()["pool2"](*a)


def kernel(x, edge_index, W1, b1, g1, be1, a1, p1W, p1b,
           W2, b2, g2, be2, a2, p2W, p2b):
    f32 = jnp.float32
    i32 = jnp.int32
    src = edge_index[0]
    dst = edge_index[1]
    pad = E_PAD - E
    src1p = jnp.concatenate([src, jnp.zeros((pad,), i32)])
    dst1p = jnp.concatenate([dst, jnp.full((pad,), N1, i32)])

    zch = jnp.zeros((CH, F), f32)

    b1r = jnp.reshape(b1, (1, F))
    g1r = jnp.reshape(g1, (1, F))
    be1r = jnp.reshape(be1, (1, F))
    a1r = jnp.reshape(a1, (1, 1))
    p1br = jnp.reshape(p1b, (1, 1))
    b2r = jnp.reshape(b2, (1, F))
    g2r = jnp.reshape(g2, (1, F))
    be2r = jnp.reshape(be2, (1, F))
    a2r = jnp.reshape(a2, (1, 1))
    p2br = jnp.reshape(p2b, (1, 1))

    dg1 = jnp.reshape(_sc_deg1(dst1p), (NC, S1))
    y1, dinv1 = _tc_dense(x, W1, jnp.transpose(dg1), N1)
    sp1 = _sc_edge_rows1(y1, src1p, dst1p, zch)
    hp1, u1 = _tc_bn(sp1, y1, dinv1, b1r, g1r, be1r, a1r, p1W, N1, S1)
    ss1 = jnp.reshape(_sc_edge_scal1(jnp.reshape(u1, (N1,)), src1p, dst1p),
                      (NC, S1))
    xs1, ni1, dp1 = _tc_rank(
        jnp.transpose(ss1), u1, dinv1, ss1,
        jnp.reshape(u1, (1, N1)), jnp.reshape(dinv1, (1, N1)),
        p1br, hp1, N1, K1, K1, S1)
    xs1p = jnp.concatenate([xs1, jnp.zeros((NP1 - N1, F), f32)])
    dp1p = jnp.concatenate([jnp.reshape(dp1, (N1,)), jnp.full((NP1 - N1,), K1, i32)])
    ni1p = jnp.concatenate([jnp.reshape(ni1, (N1,)), jnp.full((S1 - N1,), -1, i32)])
    xnewp, src2p, dst2p, dg2f = _sc_pool_remap(
        xs1p, dp1p, ni1p, src1p, dst1p)
    xnew = xnewp[:K1]
    dg2 = jnp.reshape(dg2f, (NC, S2))

    y2, dinv2 = _tc_dense(xnew, W2, jnp.transpose(dg2), K1)
    sp2 = _sc_edge_rows2(y2, src2p, dst2p, zch)
    hp2, u2 = _tc_bn(sp2, y2, dinv2, b2r, g2r, be2r, a2r, p2W, K1, S2)
    ss2 = jnp.reshape(_sc_edge_scal2(jnp.reshape(u2, (K1,)), src2p, dst2p),
                      (NC, S2))
    xs2, _, dp2 = _tc_rank(
        jnp.transpose(ss2), u2, dinv2, ss2,
        jnp.reshape(u2, (1, K1)), jnp.reshape(dinv2, (1, K1)),
        p2br, hp2, K1, K2, K2, S2)
    xs2p = jnp.concatenate([xs2, jnp.zeros((NP2 - K1, F), f32)])
    dp2p = jnp.concatenate([jnp.reshape(dp2, (K1,)), jnp.full((NP2 - K1,), K2, i32)])
    hout = _sc_pool2(xs2p, dp2p)

    h = hout[:K2]
    z = _tc_mean(hout)
    return (h, z)
